# Initial kernel scaffold; baseline (speedup 1.0000x reference)
import jax
import jax.numpy as jnp
from jax.experimental import pallas as pl


def kernel(x, label, weight, A_gen, B_gen, A_spec, B_spec):
    raise NotImplementedError("write your pallas kernel here")


# speedup vs baseline: 30.4024x; 30.4024x over previous
import functools

import jax
import jax.numpy as jnp
from jax.experimental import pallas as pl

_N = 4096
_D = 2048
_R = 64
_E = 8
_ALPHA = 2.0
_C = 640
_BN = 512


def _moe_lora_body(lab_ref, x_ref, a_ref, b_ref, o_ref):
    i = pl.program_id(0)
    x = x_ref[...]
    h = jax.lax.dot_general(
        x, a_ref[...], (((1,), (1,)), ((), ())),
        preferred_element_type=jnp.float32,
    )
    lab = lab_ref[...]
    col = jax.lax.broadcasted_iota(jnp.int32, h.shape, 1)
    keep = (col // _R == lab) | (col >= _E * _R)
    h = jnp.where(keep, h * _ALPHA, 0.0)
    out = jax.lax.dot_general(
        h, b_ref[...], (((1,), (0,)), ((), ())),
        preferred_element_type=jnp.float32,
    )
    row = jax.lax.broadcasted_iota(jnp.int32, out.shape, 0) + i * _BN
    o_ref[...] = jnp.where(row == _N - 1, 0.0, out)


@functools.partial(jax.jit, static_argnames=())
def kernel(x, label, weight, A_gen, B_gen, A_spec, B_spec):
    del weight
    lab = label.astype(jnp.int32).reshape(_N, 1)
    pad = _C - (_E + 1) * _R
    a_cat = jnp.concatenate(
        [A_spec.reshape(_E * _R, _D), A_gen,
         jnp.zeros((pad, _D), jnp.float32)], axis=0)
    b_cat = jnp.concatenate(
        [B_spec.transpose(0, 2, 1).reshape(_E * _R, _D), B_gen.T,
         jnp.zeros((pad, _D), jnp.float32)], axis=0)
    return pl.pallas_call(
        _moe_lora_body,
        grid=(_N // _BN,),
        in_specs=[
            pl.BlockSpec((_BN, 1), lambda i: (i, 0)),
            pl.BlockSpec((_BN, _D), lambda i: (i, 0)),
            pl.BlockSpec((_C, _D), lambda i: (0, 0)),
            pl.BlockSpec((_C, _D), lambda i: (0, 0)),
        ],
        out_specs=pl.BlockSpec((_BN, _D), lambda i: (i, 0)),
        out_shape=jax.ShapeDtypeStruct((_N, _D), jnp.float32),
    )(lab, x, a_cat, b_cat)
